# baseline (speedup 1.0000x reference)
import jax
import jax.numpy as jnp
from jax import lax
from jax.experimental import pallas as pl
from jax.experimental.pallas import tpu as pltpu
from jax.experimental.pallas import tpu_sc as plsc

_M = 128
_N = 32768
_L = 16
_NC = 2
_NS = 16
_NW = _NC * _NS
_TC_ROWS = 32
_SC_ROWS = _M - _TC_ROWS
_ROWS_PER_W = _SC_ROWS // _NW
_CH = 16384
_NCH = _N // _CH
_CV = _CH // _L
_BW = 512


def _compute_chunk(src, dst, carry0):
    last = jnp.full((_L,), _L - 1, jnp.int32)

    @plsc.parallel_loop(0, _CV, 1, unroll=8, carry=carry0)
    def final_carry(i, carry):
        off = (_CV - 1 - i) * _L
        v = src[pl.ds(off, _L)]
        s = plsc.cumsum(v)
        t = jnp.take_along_axis(s, last, axis=0)
        dst[pl.ds(off, _L)] = (carry + t) - s + v
        return carry + t

    return final_carry


def _sc_body(x_hbm, out_hbm, vin0, vin1, vout0, vout1, sin0, sin1, sout0, sout1):
    wid = lax.axis_index("s") * _NC + lax.axis_index("c")
    vin = (vin0, vin1)
    vout = (vout0, vout1)
    sin = (sin0, sin1)
    sout = (sout0, sout1)

    tasks = [(r, k) for r in range(_ROWS_PER_W) for k in range(_NCH - 1, -1, -1)]

    def start_in(idx):
        r, k = tasks[idx]
        b = idx % 2
        row = _TC_ROWS + wid * _ROWS_PER_W + r
        return pltpu.async_copy(
            x_hbm.at[row, pl.ds(k * _CH, _CH)], vin[b], sin[b]
        )

    pend_out = [None, None]
    pend_in = start_in(0)
    carry = jnp.zeros((_L,), jnp.float32)

    for idx, (r, k) in enumerate(tasks):
        b = idx % 2
        nxt = start_in(idx + 1) if idx + 1 < len(tasks) else None
        pend_in.wait()
        if pend_out[b] is not None:
            pend_out[b].wait()
        if k == _NCH - 1:
            carry = jnp.zeros((_L,), jnp.float32)
        carry = _compute_chunk(vin[b], vout[b], carry)
        row = wid * _ROWS_PER_W + r
        pend_out[b] = pltpu.async_copy(
            vout[b], out_hbm.at[row, pl.ds(k * _CH, _CH)], sout[b]
        )
        pend_in = nxt

    for b in (0, 1):
        if pend_out[b] is not None:
            pend_out[b].wait()


def _sc_call(x):
    mesh = plsc.VectorSubcoreMesh(core_axis_name="c", subcore_axis_name="s")
    return pl.kernel(
        _sc_body,
        out_type=jax.ShapeDtypeStruct((_SC_ROWS, _N), jnp.float32),
        mesh=mesh,
        compiler_params=pltpu.CompilerParams(needs_layout_passes=False),
        scratch_types=[
            pltpu.VMEM((_CH,), jnp.float32),
            pltpu.VMEM((_CH,), jnp.float32),
            pltpu.VMEM((_CH,), jnp.float32),
            pltpu.VMEM((_CH,), jnp.float32),
            pltpu.SemaphoreType.DMA,
            pltpu.SemaphoreType.DMA,
            pltpu.SemaphoreType.DMA,
            pltpu.SemaphoreType.DMA,
        ],
    )(x)


def _tc_body(x_ref, t_ref, o_ref, carry_ref):
    g = pl.program_id(0)

    @pl.when(g == 0)
    def _init():
        carry_ref[...] = jnp.zeros_like(carry_ref)

    b = x_ref[...]
    rev = jax.lax.dot(b, t_ref[...], preferred_element_type=jnp.float32)
    o_ref[...] = rev + carry_ref[...]
    carry_ref[...] = carry_ref[...] + rev[:, 0:1]


def _tc_call(x):
    nb = _N // _BW
    k = jax.lax.broadcasted_iota(jnp.int32, (_BW, _BW), 0)
    j = jax.lax.broadcasted_iota(jnp.int32, (_BW, _BW), 1)
    tri = (k >= j).astype(jnp.float32)
    return pl.pallas_call(
        _tc_body,
        grid=(nb,),
        in_specs=[
            pl.BlockSpec((_TC_ROWS, _BW), lambda g, nb=nb: (0, nb - 1 - g)),
            pl.BlockSpec((_BW, _BW), lambda g: (0, 0)),
        ],
        out_specs=pl.BlockSpec((_TC_ROWS, _BW), lambda g, nb=nb: (0, nb - 1 - g)),
        out_shape=jax.ShapeDtypeStruct((_TC_ROWS, _N), jnp.float32),
        scratch_shapes=[pltpu.VMEM((_TC_ROWS, 1), jnp.float32)],
    )(x, tri)


@jax.jit
def kernel(x):
    bot = _sc_call(x)
    top = _tc_call(x)
    return jnp.concatenate([top, bot], axis=0)
